# baseline (speedup 1.0000x reference)
import functools

import jax
import jax.numpy as jnp
from jax import lax
from jax.experimental import pallas as pl
from jax.experimental.pallas import tpu as pltpu
from jax.experimental.pallas import tpu_sc as plsc

_B, _S, _H = 1024, 50, 512
_SC_ROWS = 384
_TC_ROWS = _B - _SC_ROWS

_NW = 32
_SC_ROWS_PER_W = _SC_ROWS // _NW
_SC_NBUF = 4
_SC_K = 2


def _build_sc_copy():
    mesh = plsc.VectorSubcoreMesh(core_axis_name="c", subcore_axis_name="s")

    @functools.partial(
        pl.kernel,
        mesh=mesh,
        out_type=jax.ShapeDtypeStruct((_SC_ROWS, _S, _H), jnp.float32),
        scratch_types=(
            [pltpu.VMEM((1, _S, _H), jnp.float32) for _ in range(_SC_NBUF)]
            + [pltpu.SemaphoreType.DMA for _ in range(2 * _SC_NBUF)]
        ),
        compiler_params=pltpu.CompilerParams(use_tc_tiling_on_sc=True),
    )
    def sc_copy(x_hbm, o_hbm, *scratch):
        bufs = scratch[:_SC_NBUF]
        isems = scratch[_SC_NBUF:2 * _SC_NBUF]
        osems = scratch[2 * _SC_NBUF:]
        wid = lax.axis_index("s") * 2 + lax.axis_index("c")
        base = wid * _SC_ROWS_PER_W

        def in_copy(i):
            s = i % _SC_NBUF
            return pltpu.make_async_copy(
                x_hbm.at[pl.ds(base + i, 1)], bufs[s], isems[s])

        def out_copy(i):
            s = i % _SC_NBUF
            return pltpu.make_async_copy(
                bufs[s], o_hbm.at[pl.ds(base + i, 1)], osems[s])

        waited_outs = set()
        for j in range(min(_SC_K, _SC_ROWS_PER_W)):
            in_copy(j).start()
        for i in range(_SC_ROWS_PER_W):
            j = i + _SC_K
            if j < _SC_ROWS_PER_W:
                if j - _SC_NBUF >= 0:
                    out_copy(j - _SC_NBUF).wait()
                    waited_outs.add(j - _SC_NBUF)
                in_copy(j).start()
            in_copy(i).wait()
            out_copy(i).start()
        for i in range(_SC_ROWS_PER_W):
            if i not in waited_outs:
                out_copy(i).wait()

    return sc_copy


_sc_copy = _build_sc_copy()

_TC_NBUF = 8
_TC_K = 4
_TC_CHUNK = 64
_TC_NCHUNK = _TC_ROWS // _TC_CHUNK


def _tc_pipe(x_ref, o_ref, buf, in_sems, out_sems):
    def in_copy(i):
        s = i % _TC_NBUF
        return pltpu.make_async_copy(
            x_ref.at[pl.ds(_SC_ROWS + i * _TC_CHUNK, _TC_CHUNK)],
            buf.at[s], in_sems.at[s])

    def out_copy(i):
        s = i % _TC_NBUF
        return pltpu.make_async_copy(
            buf.at[s], o_ref.at[pl.ds(i * _TC_CHUNK, _TC_CHUNK)],
            out_sems.at[s])

    waited_outs = set()
    for j in range(min(_TC_K, _TC_NCHUNK)):
        in_copy(j).start()
    for i in range(_TC_NCHUNK):
        j = i + _TC_K
        if j < _TC_NCHUNK:
            if j - _TC_NBUF >= 0:
                out_copy(j - _TC_NBUF).wait()
                waited_outs.add(j - _TC_NBUF)
            in_copy(j).start()
        in_copy(i).wait()
        out_copy(i).start()
    for i in range(_TC_NCHUNK):
        if i not in waited_outs:
            out_copy(i).wait()


def _tc_copy(x):
    return pl.pallas_call(
        _tc_pipe,
        out_shape=jax.ShapeDtypeStruct((_TC_ROWS, _S, _H), x.dtype),
        in_specs=[pl.BlockSpec(memory_space=pl.ANY)],
        out_specs=pl.BlockSpec(memory_space=pl.ANY),
        scratch_shapes=[
            pltpu.VMEM((_TC_NBUF, _TC_CHUNK, _S, _H), x.dtype),
            pltpu.SemaphoreType.DMA((_TC_NBUF,)),
            pltpu.SemaphoreType.DMA((_TC_NBUF,)),
        ],
    )(x)


def kernel(x, hippocampus, neocortex):
    sc_half = _sc_copy(x)
    tc_half = _tc_copy(x)
    return jnp.concatenate([sc_half, tc_half], axis=0)
